# Initial kernel scaffold; baseline (speedup 1.0000x reference)
import jax
import jax.numpy as jnp
from jax.experimental import pallas as pl


def kernel(x, attn_mask, item_emb, lin_in_w, lin_out_w, gru_w_ih, gru_w_hh, gru_b_ih, gru_b_hh, read_w):
    raise NotImplementedError("write your pallas kernel here")


# speedup vs baseline: 1.7676x; 1.7676x over previous
import functools

import jax
import jax.numpy as jnp
from jax import lax
from jax.experimental import pallas as pl
from jax.experimental.pallas import tpu as pltpu
from jax.experimental.pallas import tpu_sc as plsc

_B, _L, _D, _V = 1024, 50, 128, 100000
_PRE_BB = 128
_GRU_BB = 64
_ATT_BB = 128
_MM_VT = 1024


def _pre_body(x_ref, uniq_ref, cself_ref, nk_ref):
    xi = x_ref[...]
    rowf = xi.astype(jnp.float32)
    vf = jnp.where(xi != 0, 1.0, 0.0)
    iot = lax.broadcasted_iota(jnp.int32, (1, _L), 1).astype(jnp.float32)
    iot3 = lax.broadcasted_iota(jnp.int32, (1, 1, _L), 2).astype(jnp.float32)
    le = jnp.where(
        lax.broadcasted_iota(jnp.int32, (1, _L, _L), 1)
        <= lax.broadcasted_iota(jnp.int32, (1, _L, _L), 2), 1.0, 0.0)
    cums = jnp.sum(vf[:, :, None] * le, axis=1)
    n = cums[:, -1:]
    cpos = cums - 1.0
    s1 = vf[:, :, None] * jnp.where(cpos[:, :, None] == iot3, 1.0, 0.0)
    seq = jnp.sum(s1 * rowf[:, :, None], axis=1)
    prev = jnp.concatenate(
        [jnp.full((seq.shape[0], 1), -1.0, jnp.float32), seq[:, :-1]], axis=1)
    mf = jnp.where((seq != prev) & (iot < n), 1.0, 0.0)
    invc = jnp.sum(mf[:, :, None] * le, axis=1)
    inv = invc - 1.0
    k = invc[:, -1:]
    s2 = mf[:, :, None] * jnp.where(inv[:, :, None] == iot3, 1.0, 0.0)
    uniqf = jnp.sum(s2 * seq[:, :, None], axis=1)
    pmask = lax.broadcasted_iota(jnp.int32, (1, _L, 1), 1).astype(jnp.float32) < n[:, :, None]
    cnt = jnp.sum(
        jnp.where(pmask & (inv[:, :, None] == iot3), 1.0, 0.0), axis=1)
    uniq_ref[...] = uniqf.astype(jnp.int32)
    cself_ref[...] = jnp.maximum(cnt - 1.0, 0.0)
    nk_ref[...] = jnp.concatenate([n, k], axis=1)


def _preprocess(x):
    grid = _B // _PRE_BB
    return pl.pallas_call(
        _pre_body,
        grid=(grid,),
        in_specs=[pl.BlockSpec((_PRE_BB, _L), lambda i: (i, 0))],
        out_specs=[
            pl.BlockSpec((_PRE_BB, _L), lambda i: (i, 0)),
            pl.BlockSpec((_PRE_BB, _L), lambda i: (i, 0)),
            pl.BlockSpec((_PRE_BB, 2), lambda i: (i, 0)),
        ],
        out_shape=[
            jax.ShapeDtypeStruct((_B, _L), jnp.int32),
            jax.ShapeDtypeStruct((_B, _L), jnp.float32),
            jax.ShapeDtypeStruct((_B, 2), jnp.float32),
        ],
    )(x)


def _gather_sc(item_emb, uniq):
    info = plsc.get_sparse_core_info()
    nc, ns = info.num_cores, info.num_subcores
    nw = nc * ns
    tot = _B * _L
    bpw = tot // nw
    ch = 80
    nch = bpw // ch
    idx3 = uniq.reshape(nw, nch, ch)
    mesh = plsc.VectorSubcoreMesh(core_axis_name="c", subcore_axis_name="s")

    @functools.partial(
        pl.kernel, mesh=mesh,
        out_type=jax.ShapeDtypeStruct((tot, _D), jnp.float32),
        scratch_types=[
            pltpu.VMEM((nch, ch), jnp.int32),
            pltpu.VMEM((ch, _D), jnp.float32),
            pltpu.VMEM((ch, _D), jnp.float32),
            pltpu.SemaphoreType.DMA,
            pltpu.SemaphoreType.DMA,
            pltpu.SemaphoreType.DMA,
            pltpu.SemaphoreType.DMA,
        ])
    def gk(table, idx, out, idx_v, buf0, buf1, sg0, sg1, so0, so1):
        wid = lax.axis_index("s") * nc + lax.axis_index("c")
        base = wid * bpw
        pltpu.sync_copy(idx.at[wid], idx_v)
        bufs = (buf0, buf1)
        gsems = (sg0, sg1)
        osems = (so0, so1)
        gcp = {}
        ocp = {}
        gcp[0] = pltpu.async_copy(table.at[idx_v.at[0]], buf0, sg0)
        for c in range(nch):
            p = c % 2
            if c + 1 < nch:
                q = (c + 1) % 2
                if c >= 1:
                    ocp[c - 1].wait()
                gcp[c + 1] = pltpu.async_copy(
                    table.at[idx_v.at[c + 1]], bufs[q], gsems[q])
            gcp[c].wait()
            ocp[c] = pltpu.async_copy(
                bufs[p], out.at[pl.ds(base + c * ch, ch)], osems[p])
        ocp[nch - 2].wait()
        ocp[nch - 1].wait()

    return gk(item_emb, idx3)


def _gru_body(node_ref, cself_ref, nrep_ref, wih_ref, whh_ref, bih_ref,
              bhh_ref, win_ref, wout_ref, h_ref):
    node = node_ref[...]
    cs = cself_ref[...]
    nr = nrep_ref[...]
    w1 = lax.dot_general(wih_ref[:, :_D], win_ref[...],
                         (((1,), (0,)), ((), ())),
                         preferred_element_type=jnp.float32)
    w2 = lax.dot_general(wih_ref[:, _D:], wout_ref[...],
                         (((1,), (0,)), ((), ())),
                         preferred_element_type=jnp.float32)
    r_rows = node.shape[0]
    zrow = jnp.zeros((1, _D), jnp.float32)
    sh_dn = jnp.concatenate([zrow, node[:-1, :]], axis=0)
    sh_up = jnp.concatenate([node[1:, :], zrow], axis=0)
    loc = lax.rem(lax.broadcasted_iota(jnp.int32, (r_rows, 1), 0), _L)
    sh_in = jnp.where(loc == 0, 0.0, sh_dn)
    sh_out = jnp.where(loc == _L - 1, 0.0, sh_up)
    a_in = sh_in + cs * node
    a_out = sh_out + cs * node
    gi = (lax.dot_general(a_in, w1, (((1,), (1,)), ((), ())),
                          preferred_element_type=jnp.float32)
          + lax.dot_general(a_out, w2, (((1,), (1,)), ((), ())),
                            preferred_element_type=jnp.float32)
          + bih_ref[0:1, :])
    gh = lax.dot_general(node, whh_ref[...], (((1,), (1,)), ((), ())),
                         preferred_element_type=jnp.float32) + bhh_ref[0:1, :]
    r = jax.nn.sigmoid(gi[:, :_D] + gh[:, :_D])
    z = jax.nn.sigmoid(gi[:, _D:2 * _D] + gh[:, _D:2 * _D])
    nn_ = jnp.tanh(gi[:, 2 * _D:] + r * gh[:, 2 * _D:])
    h = (1.0 - z) * nn_ + z * node
    h_ref[...] = jnp.where(nr >= 2.0, h, node)


def _gru(node, cself_col, nrep, wih, whh, bih8, bhh8, win, wout):
    rows = _B * _L
    rblk = _GRU_BB * _L
    grid = rows // rblk
    full2 = lambda shape: pl.BlockSpec(shape, lambda i: (0, 0))
    return pl.pallas_call(
        _gru_body,
        grid=(grid,),
        in_specs=[
            pl.BlockSpec((rblk, _D), lambda i: (i, 0)),
            pl.BlockSpec((rblk, 1), lambda i: (i, 0)),
            pl.BlockSpec((rblk, 1), lambda i: (i, 0)),
            full2((3 * _D, 2 * _D)),
            full2((3 * _D, _D)),
            full2((8, 3 * _D)),
            full2((8, 3 * _D)),
            full2((_D, _D)),
            full2((_D, _D)),
        ],
        out_specs=pl.BlockSpec((rblk, _D), lambda i: (i, 0)),
        out_shape=jax.ShapeDtypeStruct((rows, _D), jnp.float32),
    )(node, cself_col, nrep, wih, whh, bih8, bhh8, win, wout)


def _att_body(h_ref, nk_ref, readw_ref, s_ref):
    h = h_ref[...]
    n = nk_ref[:, 0:1]
    k = nk_ref[:, 1:2]
    iot = lax.broadcasted_iota(jnp.int32, (1, _L), 1).astype(jnp.float32)
    oh_last = jnp.where(iot == (k - 1.0), 1.0, 0.0)
    q_pre = jnp.sum(oh_last[:, :, None] * h, axis=1)
    q = lax.dot_general(q_pre, readw_ref[...], (((1,), (1,)), ((), ())),
                        preferred_element_type=jnp.float32)
    logits = jnp.sum(h * q[:, None, :], axis=2)
    logits = jnp.where(iot < k, logits, -1e30)
    mx = jnp.max(logits, axis=1, keepdims=True)
    e = jnp.exp(logits - mx)
    att = e / jnp.sum(e, axis=1, keepdims=True)
    s = jnp.sum(att[:, :, None] * h, axis=1)
    s_ref[...] = jnp.where(n > 0.0, s, 0.0)


def _attention(h3, nk, read_w):
    grid = _B // _ATT_BB
    return pl.pallas_call(
        _att_body,
        grid=(grid,),
        in_specs=[
            pl.BlockSpec((_ATT_BB, _L, _D), lambda i: (i, 0, 0)),
            pl.BlockSpec((_ATT_BB, 2), lambda i: (i, 0)),
            pl.BlockSpec((_D, _D), lambda i: (0, 0)),
        ],
        out_specs=pl.BlockSpec((_ATT_BB, _D), lambda i: (i, 0)),
        out_shape=jax.ShapeDtypeStruct((_B, _D), jnp.float32),
    )(h3, nk, read_w)


def _mm_body(s_ref, emb_ref, o_ref):
    o_ref[...] = lax.dot_general(s_ref[...], emb_ref[...],
                                 (((1,), (1,)), ((), ())),
                                 preferred_element_type=jnp.float32)


def _project(s, item_emb):
    grid = pl.cdiv(_V, _MM_VT)
    return pl.pallas_call(
        _mm_body,
        grid=(grid,),
        in_specs=[
            pl.BlockSpec((_B, _D), lambda i: (0, 0)),
            pl.BlockSpec((_MM_VT, _D), lambda i: (i, 0)),
        ],
        out_specs=pl.BlockSpec((_B, _MM_VT), lambda i: (0, i)),
        out_shape=jax.ShapeDtypeStruct((_B, _V), jnp.float32),
    )(s, item_emb)


def kernel(x, attn_mask, item_emb, lin_in_w, lin_out_w, gru_w_ih, gru_w_hh,
           gru_b_ih, gru_b_hh, read_w):
    del attn_mask
    uniq, cself, nk = _preprocess(x)
    node = _gather_sc(item_emb, uniq)
    cself_col = cself.reshape(_B * _L, 1)
    nrep = jnp.repeat(nk[:, 0:1], _L, axis=0)
    bih8 = jnp.tile(gru_b_ih.reshape(1, -1), (8, 1))
    bhh8 = jnp.tile(gru_b_hh.reshape(1, -1), (8, 1))
    h2d = _gru(node, cself_col, nrep, gru_w_ih, gru_w_hh, bih8, bhh8,
               lin_in_w, lin_out_w)
    h3 = h2d.reshape(_B, _L, _D)
    s = _attention(h3, nk, read_w)
    return _project(s, item_emb)
